# Initial kernel scaffold; baseline (speedup 1.0000x reference)
import jax
import jax.numpy as jnp
from jax.experimental import pallas as pl


def kernel(kpts, disp, features_fixed):
    raise NotImplementedError("write your pallas kernel here")


# speedup vs baseline: 3.5254x; 3.5254x over previous
import functools

import jax
import jax.numpy as jnp
import numpy as np
from jax.experimental import pallas as pl

_DIN = 48
_DOUT = 192
_DT = 8


def _interp_matrix(n_in: int, n_out: int) -> np.ndarray:
    m = np.zeros((n_in, n_out), dtype=np.float32)
    scale = n_in / n_out
    for o in range(n_out):
        c = (o + 0.5) * scale - 0.5
        i0 = int(np.floor(c))
        t = c - i0
        m[min(max(i0, 0), n_in - 1), o] += 1.0 - t
        m[min(max(i0 + 1, 0), n_in - 1), o] += t
    return m


def _body(a_ref, mht_ref, mw_ref, o_ref):
    i = pl.program_id(0)
    mht = mht_ref[...]
    mw = mw_ref[...]
    for k in range(_DT):
        od = (i * _DT + k).astype(jnp.float32)
        coord = od * (_DIN / _DOUT) - 0.375
        i0f = jnp.floor(coord)
        t = coord - i0f
        i0 = jnp.clip(i0f.astype(jnp.int32), 0, _DIN - 1)
        i1 = jnp.clip(i0f.astype(jnp.int32) + 1, 0, _DIN - 1)
        for c in range(3):
            p0 = a_ref[c, i0]
            p1 = a_ref[c, i1]
            pd = (1.0 - t) * p0 + t * p1
            s2 = jnp.dot(mht, pd, preferred_element_type=jnp.float32)
            s3 = jnp.dot(s2, mw, preferred_element_type=jnp.float32)
            o_ref[c, k] = s3


@jax.jit
def _upsample(disp):
    a = jnp.transpose(jnp.reshape(disp, (_DIN, _DIN, _DIN, 3)), (3, 0, 1, 2))
    mw = jnp.asarray(_interp_matrix(_DIN, _DOUT))
    mht = mw.T
    out = pl.pallas_call(
        _body,
        grid=(_DOUT // _DT,),
        in_specs=[
            pl.BlockSpec((3, _DIN, _DIN, _DIN), lambda i: (0, 0, 0, 0)),
            pl.BlockSpec((_DOUT, _DIN), lambda i: (0, 0)),
            pl.BlockSpec((_DIN, _DOUT), lambda i: (0, 0)),
        ],
        out_specs=pl.BlockSpec((3, _DT, _DOUT, _DOUT), lambda i: (0, i, 0, 0)),
        out_shape=jax.ShapeDtypeStruct((3, _DOUT, _DOUT, _DOUT), jnp.float32),
    )(a, mht, mw)
    return jnp.reshape(out, (1, 3, _DOUT, _DOUT, _DOUT))


def kernel(kpts, disp, features_fixed):
    del kpts, features_fixed
    return _upsample(disp)


# speedup vs baseline: 5.2201x; 1.4807x over previous
import functools

import jax
import jax.numpy as jnp
import numpy as np
from jax.experimental import pallas as pl

_DIN = 48
_DOUT = 192
_DT = 16
_NPLANES = _DT // 4 + 2


def _interp_matrix(n_in: int, n_out: int) -> np.ndarray:
    m = np.zeros((n_in, n_out), dtype=np.float32)
    scale = n_in / n_out
    for o in range(n_out):
        c = (o + 0.5) * scale - 0.5
        i0 = int(np.floor(c))
        t = c - i0
        m[min(max(i0, 0), n_in - 1), o] += 1.0 - t
        m[min(max(i0 + 1, 0), n_in - 1), o] += t
    return m


def _body(a_ref, mht_ref, mw_ref, o_ref):
    i = pl.program_id(0)
    mht = mht_ref[...]
    mw = mw_ref[...]
    d0 = i * (_DT // 4) - 1
    u = []
    for c in range(3):
        uc = []
        for j in range(_NPLANES):
            dj = jnp.clip(d0 + j, 0, _DIN - 1)
            s2 = jnp.dot(mht, a_ref[c, dj], preferred_element_type=jnp.float32)
            uc.append(jnp.dot(s2, mw, preferred_element_type=jnp.float32))
        u.append(uc)
    for k in range(_DT):
        i0rel = (2 * k - 3) // 8
        frac = k * 0.25 - 0.375 - i0rel
        j0 = i0rel + 1
        for c in range(3):
            o_ref[c, k] = (1.0 - frac) * u[c][j0] + frac * u[c][j0 + 1]


@jax.jit
def _upsample(disp):
    a = jnp.transpose(jnp.reshape(disp, (_DIN, _DIN, _DIN, 3)), (3, 0, 1, 2))
    mw = jnp.asarray(_interp_matrix(_DIN, _DOUT))
    mht = mw.T
    out = pl.pallas_call(
        _body,
        grid=(_DOUT // _DT,),
        in_specs=[
            pl.BlockSpec((3, _DIN, _DIN, _DIN), lambda i: (0, 0, 0, 0)),
            pl.BlockSpec((_DOUT, _DIN), lambda i: (0, 0)),
            pl.BlockSpec((_DIN, _DOUT), lambda i: (0, 0)),
        ],
        out_specs=pl.BlockSpec((3, _DT, _DOUT, _DOUT), lambda i: (0, i, 0, 0)),
        out_shape=jax.ShapeDtypeStruct((3, _DOUT, _DOUT, _DOUT), jnp.float32),
    )(a, mht, mw)
    return jnp.reshape(out, (1, 3, _DOUT, _DOUT, _DOUT))


def kernel(kpts, disp, features_fixed):
    del kpts, features_fixed
    return _upsample(disp)
